# baseline (speedup 1.0000x reference)
import functools

import jax
import jax.numpy as jnp
from jax import lax
from jax.experimental import pallas as pl
from jax.experimental.pallas import tpu as pltpu
from jax.experimental.pallas import tpu_sc as plsc

NP = 51200
CHUNK = 128
W = 32
BLK = 1024
N_TILES = 16
ROWS_PER_TILE = NP // N_TILES
COPY_PER_TILE = ROWS_PER_TILE // CHUNK
ECHUNKS = 6250


XCH = NP // CHUNK
XPT = (XCH + 31) // 32


def _embed_call(xidx, e0, e1, e2):
    mesh = plsc.VectorSubcoreMesh(core_axis_name="c", subcore_axis_name="s")

    @functools.partial(
        pl.kernel, mesh=mesh,
        out_type=jax.ShapeDtypeStruct((3, NP, 64), jnp.float32),
        compiler_params=pltpu.CompilerParams(use_tc_tiling_on_sc=False),
        scratch_types=[
            pltpu.VMEM((XPT, CHUNK), jnp.int32),
            pltpu.VMEM((2, CHUNK, 64), jnp.float32),
            pltpu.SemaphoreType.DMA,
            pltpu.SemaphoreType.DMA,
        ],
    )
    def k(xidx_hbm, e0_hbm, e1_hbm, e2_hbm, out_hbm, idx_v, rows_v,
          sem_g, sem_w):
        c = lax.axis_index("c")
        s = lax.axis_index("s")
        wid = s * 2 + c
        base = wid * XPT
        tabs = (e0_hbm, e1_hbm, e2_hbm)
        zdesc = lambda p: pltpu.make_async_copy(
            out_hbm.at[0, pl.ds(0, CHUNK)], rows_v.at[p], sem_w)

        for f in range(3):
            tab = tabs[f]
            pltpu.sync_copy(xidx_hbm.at[f, pl.ds(base, XPT)], idx_v)

            def gather(p, j):
                pltpu.async_copy(tab.at[idx_v.at[j]], rows_v.at[p], sem_g)
                pltpu.make_async_copy(out_hbm.at[0, pl.ds(0, CHUNK)],
                                      rows_v.at[p], sem_g).wait()

            def writeout(p, j):
                ch = jnp.minimum(base + j, XCH - 1)
                pltpu.async_copy(rows_v.at[p],
                                 out_hbm.at[f, pl.ds(ch * CHUNK, CHUNK)],
                                 sem_w)

            gather(0, 0)
            writeout(0, 0)
            gather(1, 1)
            writeout(1, 1)

            def body(m, _, f=f):
                j = 2 + 2 * m
                zdesc(0).wait()
                gather(0, j)
                writeout(0, j)
                zdesc(1).wait()
                gather(1, j + 1)
                writeout(1, j + 1)
                return 0

            lax.fori_loop(0, (XPT - 3) // 2, body, 0)
            zdesc(0).wait()
            gather(0, XPT - 1)
            writeout(0, XPT - 1)
            zdesc(1).wait()
            zdesc(0).wait()

    return k(xidx, e0, e1, e2)


NB = 2
EC = 6400
EGROUPS = EC // NB
G_PER_TILE = EGROUPS // N_TILES


def _agg_call(y2, ei4, zrows):
    mesh = plsc.VectorSubcoreMesh(core_axis_name="c", subcore_axis_name="s")

    @functools.partial(
        pl.kernel, mesh=mesh,
        out_type=jax.ShapeDtypeStruct((2, NP, W), jnp.float32),
        compiler_params=pltpu.CompilerParams(use_tc_tiling_on_sc=False),
        scratch_types=[
            pltpu.VMEM((4, NB, 2, CHUNK), jnp.int32),
            pltpu.VMEM((4, NB, CHUNK), jnp.int32),
            pltpu.VMEM((2, NB, CHUNK, W), jnp.float32),
            pltpu.VMEM((CHUNK, W), jnp.float32),
            pltpu.VMEM_SHARED((NP, W), jnp.float32),
            pltpu.SemaphoreType.DMA,
            pltpu.SemaphoreType.DMA,
            pltpu.SemaphoreType.DMA,
            pltpu.SemaphoreType.DMA,
        ],
    )
    def k(y2_hbm, ei_hbm, z_hbm, out_hbm,
          idx_v, soff_v, rows_v, stage_v, acc_sh, sem_i0, sem_i1,
          sem_g, sem_s):
        c = lax.axis_index("c")
        s = lax.axis_index("s")
        coff = c * NP
        pltpu.sync_copy(z_hbm.at[0], stage_v)
        for m in range(COPY_PER_TILE):
            pltpu.sync_copy(stage_v,
                            acc_sh.at[pl.ds(s * ROWS_PER_TILE + m * CHUNK,
                                            CHUNK)])
        plsc.subcore_barrier()
        gbase = s * G_PER_TILE

        def fire_idx(q, g):
            sem = sem_i0 if q % 2 == 0 else sem_i1
            pltpu.async_copy(ei_hbm.at[gbase + g], idx_v.at[q], sem)

        def run_group(p, q, g, drain, prefetch):
            if drain:
                pltpu.make_async_copy(z_hbm, rows_v.at[p], sem_s).wait()
            sem = sem_i0 if q % 2 == 0 else sem_i1
            pltpu.make_async_copy(ei_hbm.at[0], idx_v.at[q], sem).wait()
            if prefetch:
                fire_idx((q + 2) % 4, g + 2)
            for b in range(NB):
                for i in range(CHUNK // 16):
                    sl = pl.ds(i * 16, 16)
                    soff_v[q, b, sl] = idx_v[q, b, 0, sl] + coff
            for b in range(NB):
                pltpu.async_copy(y2_hbm.at[soff_v.at[q, b]],
                                 rows_v.at[p, b], sem_g)
            pltpu.make_async_copy(z_hbm, rows_v.at[p], sem_g).wait()
            for b in range(NB):
                pltpu.async_copy(rows_v.at[p, b],
                                 acc_sh.at[idx_v.at[q, b, 1]], sem_s,
                                 add=True)

        fire_idx(0, 0)
        fire_idx(1, 1)
        run_group(0, 0, 0, False, True)
        run_group(1, 1, 1, False, True)

        def body(m, _):
            g = 2 + 4 * m
            run_group(0, 2, g, True, True)
            run_group(1, 3, g + 1, True, True)
            run_group(0, 0, g + 2, True, True)
            run_group(1, 1, g + 3, True, True)
            return 0

        lax.fori_loop(0, (G_PER_TILE - 4) // 4, body, 0)
        run_group(0, 2, G_PER_TILE - 2, True, False)
        run_group(1, 3, G_PER_TILE - 1, True, False)
        pltpu.make_async_copy(z_hbm, rows_v.at[0], sem_s).wait()
        pltpu.make_async_copy(z_hbm, rows_v.at[1], sem_s).wait()
        plsc.subcore_barrier()
        for m in range(COPY_PER_TILE):
            sl = pl.ds(s * ROWS_PER_TILE + m * CHUNK, CHUNK)
            pltpu.sync_copy(acc_sh.at[sl], stage_v)
            pltpu.sync_copy(stage_v, out_hbm.at[c, sl])

    return k(y2, ei4, zrows)


NBC = 8
CGROUPS = EC // NBC
CG_PER_TILE = CGROUPS // 2 // N_TILES


def _cnt_call(ei8, onechunk, z8):
    mesh = plsc.VectorSubcoreMesh(core_axis_name="c", subcore_axis_name="s")
    gpt = CG_PER_TILE

    @functools.partial(
        pl.kernel, mesh=mesh,
        out_type=jax.ShapeDtypeStruct((2, NP, W), jnp.float32),
        compiler_params=pltpu.CompilerParams(use_tc_tiling_on_sc=False),
        scratch_types=[
            pltpu.VMEM((4, NBC, 2, CHUNK), jnp.int32),
            pltpu.VMEM((CHUNK, W), jnp.float32),
            pltpu.VMEM((CHUNK, W), jnp.float32),
            pltpu.VMEM((NBC, CHUNK, W), jnp.float32),
            pltpu.VMEM_SHARED((NP, W), jnp.float32),
            pltpu.SemaphoreType.DMA,
            pltpu.SemaphoreType.DMA,
            pltpu.SemaphoreType.DMA,
        ],
    )
    def k(ei_hbm, one_hbm, z_hbm, out_hbm,
          idx_v, ones_v, stage_v, drain_v, acc_sh, sem_i0, sem_i1, sem_s):
        c = lax.axis_index("c")
        s = lax.axis_index("s")
        pltpu.sync_copy(z_hbm.at[0], stage_v)
        for m in range(COPY_PER_TILE):
            pltpu.sync_copy(stage_v,
                            acc_sh.at[pl.ds(s * ROWS_PER_TILE + m * CHUNK,
                                            CHUNK)])
        pltpu.sync_copy(one_hbm, ones_v)
        plsc.subcore_barrier()
        gbase = (c * N_TILES + s) * gpt

        def fire_idx(q, g):
            sem = sem_i0 if q % 2 == 0 else sem_i1
            pltpu.async_copy(ei_hbm.at[gbase + g], idx_v.at[q], sem)

        def run_group(q, g, drain, prefetch):
            if drain:
                pltpu.make_async_copy(z_hbm, drain_v, sem_s).wait()
            sem = sem_i0 if q % 2 == 0 else sem_i1
            pltpu.make_async_copy(ei_hbm.at[0], idx_v.at[q], sem).wait()
            if prefetch:
                fire_idx((q + 2) % 4, g + 2)
            for b in range(NBC):
                pltpu.async_copy(ones_v, acc_sh.at[idx_v.at[q, b, 1]],
                                 sem_s, add=True)

        fire_idx(0, 0)
        fire_idx(1, 1)
        run_group(0, 0, False, True)
        run_group(1, 1, False, True)

        def body(m, _):
            g = 2 + 4 * m
            run_group(2, g, True, True)
            run_group(3, g + 1, True, True)
            run_group(0, g + 2, True, True)
            run_group(1, g + 3, True, True)
            return 0

        lax.fori_loop(0, (gpt - 5) // 4, body, 0)
        run_group(2, gpt - 3, True, True)
        run_group(3, gpt - 2, True, False)
        run_group(0, gpt - 1, True, False)
        pltpu.make_async_copy(z_hbm, drain_v, sem_s).wait()
        pltpu.make_async_copy(z_hbm, drain_v, sem_s).wait()
        plsc.subcore_barrier()
        for m in range(COPY_PER_TILE):
            sl = pl.ds(s * ROWS_PER_TILE + m * CHUNK, CHUNK)
            pltpu.sync_copy(acc_sh.at[sl], stage_v)
            pltpu.sync_copy(stage_v, out_hbm.at[c, sl])

    return k(ei8, onechunk, z8)


def _layer0_tc(x_ref, wrel_ref, wroot_ref, b_ref, y2_ref, r_ref):
    x0 = x_ref[0]
    x1 = x_ref[1]
    x2 = x_ref[2]
    wr = wrel_ref[...]
    wt = wroot_ref[...]
    dot = functools.partial(jnp.dot, preferred_element_type=jnp.float32)
    y = dot(x0, wr[0:64]) + dot(x1, wr[64:128]) + dot(x2, wr[128:192])
    r = dot(x0, wt[0:64]) + dot(x1, wt[64:128]) + dot(x2, wt[128:192])
    y2_ref[0] = y[:, :32]
    y2_ref[1] = y[:, 32:]
    r_ref[...] = r + b_ref[...]


def _elu_mean(s_ref, cnt_ref, r_ref):
    cnt = cnt_ref[0, :, 0:1] + cnt_ref[1, :, 0:1]
    inv = 1.0 / jnp.maximum(cnt, 1.0)
    ssum = jnp.concatenate([s_ref[0], s_ref[1]], axis=1)
    h = ssum * inv + r_ref[...]
    return jnp.where(h > 0, h, jnp.exp(h) - 1.0)


def _layer1_tc(s_ref, cnt_ref, r0_ref, wrel_ref, wroot_ref, b_ref,
               y2_ref, r_ref):
    h = _elu_mean(s_ref, cnt_ref, r0_ref)
    dot = functools.partial(jnp.dot, preferred_element_type=jnp.float32)
    y = dot(h, wrel_ref[...])
    y2_ref[0] = y[:, :32]
    y2_ref[1] = y[:, 32:]
    r_ref[...] = dot(h, wroot_ref[...]) + b_ref[...]


def _final_tc(s_ref, cnt_ref, r1_ref, wout_ref, bout_ref, out_ref):
    h = _elu_mean(s_ref, cnt_ref, r1_ref)
    out_ref[...] = jnp.dot(h, wout_ref[...],
                           preferred_element_type=jnp.float32) + bout_ref[...]


def _layer0_call(x, wrel, wroot, b):
    return pl.pallas_call(
        _layer0_tc,
        grid=(NP // BLK,),
        in_specs=[
            pl.BlockSpec((3, BLK, 64), lambda i: (0, i, 0)),
            pl.BlockSpec((192, 64), lambda i: (0, 0)),
            pl.BlockSpec((192, 64), lambda i: (0, 0)),
            pl.BlockSpec((1, 64), lambda i: (0, 0)),
        ],
        out_specs=[
            pl.BlockSpec((2, BLK, W), lambda i: (0, i, 0)),
            pl.BlockSpec((BLK, 64), lambda i: (i, 0)),
        ],
        out_shape=[
            jax.ShapeDtypeStruct((2, NP, W), jnp.float32),
            jax.ShapeDtypeStruct((NP, 64), jnp.float32),
        ],
    )(x, wrel, wroot, b)


def _layer1_call(s, cnt, r0, wrel, wroot, b):
    return pl.pallas_call(
        _layer1_tc,
        grid=(NP // BLK,),
        in_specs=[
            pl.BlockSpec((2, BLK, W), lambda i: (0, i, 0)),
            pl.BlockSpec((2, BLK, W), lambda i: (0, i, 0)),
            pl.BlockSpec((BLK, 64), lambda i: (i, 0)),
            pl.BlockSpec((64, 64), lambda i: (0, 0)),
            pl.BlockSpec((64, 64), lambda i: (0, 0)),
            pl.BlockSpec((1, 64), lambda i: (0, 0)),
        ],
        out_specs=[
            pl.BlockSpec((2, BLK, W), lambda i: (0, i, 0)),
            pl.BlockSpec((BLK, 64), lambda i: (i, 0)),
        ],
        out_shape=[
            jax.ShapeDtypeStruct((2, NP, W), jnp.float32),
            jax.ShapeDtypeStruct((NP, 64), jnp.float32),
        ],
    )(s, cnt, r0, wrel, wroot, b)


def _final_call(s, cnt, r1, wout, bout):
    return pl.pallas_call(
        _final_tc,
        grid=(NP // BLK,),
        in_specs=[
            pl.BlockSpec((2, BLK, W), lambda i: (0, i, 0)),
            pl.BlockSpec((2, BLK, W), lambda i: (0, i, 0)),
            pl.BlockSpec((BLK, 64), lambda i: (i, 0)),
            pl.BlockSpec((64, 32), lambda i: (0, 0)),
            pl.BlockSpec((1, 32), lambda i: (0, 0)),
        ],
        out_specs=pl.BlockSpec((BLK, 32), lambda i: (i, 0)),
        out_shape=jax.ShapeDtypeStruct((NP, 32), jnp.float32),
    )(s, cnt, r1, wout, bout)


def kernel(x, edge_index, emb0, emb1, emb2, W_rel0, W_root0, b0,
           W_rel1, W_root1, b1, W_out, b_out):
    n = x.shape[0]
    e = edge_index.shape[1]

    xidx = jnp.pad(x.astype(jnp.int32).T, ((0, 0), (0, NP - n)))
    xidx = xidx.reshape(3, NP // CHUNK, CHUNK)
    xidx = jnp.concatenate(
        [xidx, jnp.broadcast_to(xidx[:, -1:], (3, 32 * XPT - NP // CHUNK,
                                               CHUNK))], axis=1)
    pad_e = EC * CHUNK - e
    fill = jnp.arange(pad_e, dtype=jnp.int32)
    srcp = jnp.concatenate([edge_index[0].astype(jnp.int32), fill % n])
    dstp = jnp.concatenate([edge_index[1].astype(jnp.int32),
                            n + fill % (NP - n)])
    ei = jnp.stack([srcp.reshape(EC, CHUNK), dstp.reshape(EC, CHUNK)],
                   axis=1)
    ei4 = ei.reshape(EC // NB, NB, 2, CHUNK)
    ei8 = ei.reshape(CGROUPS, NBC, 2, CHUNK)
    zrows = jnp.zeros((NB, CHUNK, W), jnp.float32)
    z8 = jnp.zeros((NBC, CHUNK, W), jnp.float32)
    onechunk = jnp.zeros((CHUNK, W), jnp.float32).at[:, 0].set(1.0)

    X = _embed_call(xidx, emb0, emb1, emb2)
    cnt = _cnt_call(ei8, onechunk, z8)

    y20, r0 = _layer0_call(X, W_rel0, W_root0, b0.reshape(1, 64))
    s0 = _agg_call(y20.reshape(2 * NP, W), ei4, zrows)

    y21, r1 = _layer1_call(s0, cnt, r0, W_rel1, W_root1, b1.reshape(1, 64))
    s1 = _agg_call(y21.reshape(2 * NP, W), ei4, zrows)

    logits = _final_call(s1, cnt, r1, W_out, b_out.reshape(1, 32))
    return logits[:n]


# speedup vs baseline: 1.0012x; 1.0012x over previous
import functools

import jax
import jax.numpy as jnp
from jax import lax
from jax.experimental import pallas as pl
from jax.experimental.pallas import tpu as pltpu
from jax.experimental.pallas import tpu_sc as plsc

NP = 51200
CHUNK = 128
W = 32
BLK = 1024
N_TILES = 16
ROWS_PER_TILE = NP // N_TILES
COPY_PER_TILE = ROWS_PER_TILE // CHUNK
ECHUNKS = 6250


XCH = NP // CHUNK
XPT = (XCH + 31) // 32


def _embed_call(xidx, e0, e1, e2):
    mesh = plsc.VectorSubcoreMesh(core_axis_name="c", subcore_axis_name="s")

    @functools.partial(
        pl.kernel, mesh=mesh,
        out_type=jax.ShapeDtypeStruct((3, NP, 64), jnp.float32),
        compiler_params=pltpu.CompilerParams(use_tc_tiling_on_sc=False),
        scratch_types=[
            pltpu.VMEM((XPT, CHUNK), jnp.int32),
            pltpu.VMEM((2, CHUNK, 64), jnp.float32),
            pltpu.SemaphoreType.DMA,
            pltpu.SemaphoreType.DMA,
        ],
    )
    def k(xidx_hbm, e0_hbm, e1_hbm, e2_hbm, out_hbm, idx_v, rows_v,
          sem_g, sem_w):
        c = lax.axis_index("c")
        s = lax.axis_index("s")
        wid = s * 2 + c
        base = wid * XPT
        tabs = (e0_hbm, e1_hbm, e2_hbm)
        zdesc = lambda p: pltpu.make_async_copy(
            out_hbm.at[0, pl.ds(0, CHUNK)], rows_v.at[p], sem_w)

        for f in range(3):
            tab = tabs[f]
            pltpu.sync_copy(xidx_hbm.at[f, pl.ds(base, XPT)], idx_v)

            def gather(p, j):
                pltpu.async_copy(tab.at[idx_v.at[j]], rows_v.at[p], sem_g)
                pltpu.make_async_copy(out_hbm.at[0, pl.ds(0, CHUNK)],
                                      rows_v.at[p], sem_g).wait()

            def writeout(p, j):
                ch = jnp.minimum(base + j, XCH - 1)
                pltpu.async_copy(rows_v.at[p],
                                 out_hbm.at[f, pl.ds(ch * CHUNK, CHUNK)],
                                 sem_w)

            for j in range(XPT):
                p = j % 2
                if j >= 2:
                    zdesc(p).wait()
                gather(p, j)
                writeout(p, j)
            zdesc((XPT - 2) % 2).wait()
            zdesc((XPT - 1) % 2).wait()

    return k(xidx, e0, e1, e2)


NB = 2
EC = 6400
EGROUPS = EC // NB
G_PER_TILE = EGROUPS // N_TILES


def _agg_call(y2, ei4, zrows):
    mesh = plsc.VectorSubcoreMesh(core_axis_name="c", subcore_axis_name="s")

    @functools.partial(
        pl.kernel, mesh=mesh,
        out_type=jax.ShapeDtypeStruct((2, NP, W), jnp.float32),
        compiler_params=pltpu.CompilerParams(use_tc_tiling_on_sc=False),
        scratch_types=[
            pltpu.VMEM((4, NB, 2, CHUNK), jnp.int32),
            pltpu.VMEM((4, NB, CHUNK), jnp.int32),
            pltpu.VMEM((2, NB, CHUNK, W), jnp.float32),
            pltpu.VMEM((CHUNK, W), jnp.float32),
            pltpu.VMEM_SHARED((NP, W), jnp.float32),
            pltpu.SemaphoreType.DMA,
            pltpu.SemaphoreType.DMA,
            pltpu.SemaphoreType.DMA,
            pltpu.SemaphoreType.DMA,
        ],
    )
    def k(y2_hbm, ei_hbm, z_hbm, out_hbm,
          idx_v, soff_v, rows_v, stage_v, acc_sh, sem_i0, sem_i1,
          sem_g, sem_s):
        c = lax.axis_index("c")
        s = lax.axis_index("s")
        coff = c * NP
        pltpu.sync_copy(z_hbm.at[0], stage_v)
        for m in range(COPY_PER_TILE):
            pltpu.sync_copy(stage_v,
                            acc_sh.at[pl.ds(s * ROWS_PER_TILE + m * CHUNK,
                                            CHUNK)])
        plsc.subcore_barrier()
        gbase = s * G_PER_TILE

        def fire_idx(q, g):
            sem = sem_i0 if q % 2 == 0 else sem_i1
            pltpu.async_copy(ei_hbm.at[gbase + g], idx_v.at[q], sem)

        def run_group(p, q, g, drain, prefetch):
            if drain:
                pltpu.make_async_copy(z_hbm, rows_v.at[p], sem_s).wait()
            sem = sem_i0 if q % 2 == 0 else sem_i1
            pltpu.make_async_copy(ei_hbm.at[0], idx_v.at[q], sem).wait()
            if prefetch:
                fire_idx((q + 2) % 4, g + 2)
            for b in range(NB):
                for i in range(CHUNK // 16):
                    sl = pl.ds(i * 16, 16)
                    soff_v[q, b, sl] = idx_v[q, b, 0, sl] + coff
            for b in range(NB):
                pltpu.async_copy(y2_hbm.at[soff_v.at[q, b]],
                                 rows_v.at[p, b], sem_g)
            pltpu.make_async_copy(z_hbm, rows_v.at[p], sem_g).wait()
            for b in range(NB):
                pltpu.async_copy(rows_v.at[p, b],
                                 acc_sh.at[idx_v.at[q, b, 1]], sem_s,
                                 add=True)

        fire_idx(0, 0)
        fire_idx(1, 1)
        run_group(0, 0, 0, False, True)
        run_group(1, 1, 1, False, True)

        def body(m, _):
            g = 2 + 4 * m
            run_group(0, 2, g, True, True)
            run_group(1, 3, g + 1, True, True)
            run_group(0, 0, g + 2, True, True)
            run_group(1, 1, g + 3, True, True)
            return 0

        lax.fori_loop(0, (G_PER_TILE - 4) // 4, body, 0)
        run_group(0, 2, G_PER_TILE - 2, True, False)
        run_group(1, 3, G_PER_TILE - 1, True, False)
        pltpu.make_async_copy(z_hbm, rows_v.at[0], sem_s).wait()
        pltpu.make_async_copy(z_hbm, rows_v.at[1], sem_s).wait()
        plsc.subcore_barrier()
        for m in range(COPY_PER_TILE):
            sl = pl.ds(s * ROWS_PER_TILE + m * CHUNK, CHUNK)
            pltpu.sync_copy(acc_sh.at[sl], stage_v)
            pltpu.sync_copy(stage_v, out_hbm.at[c, sl])

    return k(y2, ei4, zrows)


NBC = 8
CGROUPS = EC // NBC
CG_PER_TILE = CGROUPS // 2 // N_TILES


def _cnt_call(ei8, onechunk, z8):
    mesh = plsc.VectorSubcoreMesh(core_axis_name="c", subcore_axis_name="s")
    gpt = CG_PER_TILE

    @functools.partial(
        pl.kernel, mesh=mesh,
        out_type=jax.ShapeDtypeStruct((2, NP, W), jnp.float32),
        compiler_params=pltpu.CompilerParams(use_tc_tiling_on_sc=False),
        scratch_types=[
            pltpu.VMEM((4, NBC, 2, CHUNK), jnp.int32),
            pltpu.VMEM((CHUNK, W), jnp.float32),
            pltpu.VMEM((CHUNK, W), jnp.float32),
            pltpu.VMEM((NBC, CHUNK, W), jnp.float32),
            pltpu.VMEM_SHARED((NP, W), jnp.float32),
            pltpu.SemaphoreType.DMA,
            pltpu.SemaphoreType.DMA,
            pltpu.SemaphoreType.DMA,
        ],
    )
    def k(ei_hbm, one_hbm, z_hbm, out_hbm,
          idx_v, ones_v, stage_v, drain_v, acc_sh, sem_i0, sem_i1, sem_s):
        c = lax.axis_index("c")
        s = lax.axis_index("s")
        pltpu.sync_copy(z_hbm.at[0], stage_v)
        for m in range(COPY_PER_TILE):
            pltpu.sync_copy(stage_v,
                            acc_sh.at[pl.ds(s * ROWS_PER_TILE + m * CHUNK,
                                            CHUNK)])
        pltpu.sync_copy(one_hbm, ones_v)
        plsc.subcore_barrier()
        gbase = (c * N_TILES + s) * gpt

        def fire_idx(q, g):
            sem = sem_i0 if q % 2 == 0 else sem_i1
            pltpu.async_copy(ei_hbm.at[gbase + g], idx_v.at[q], sem)

        def run_group(q, g, drain, prefetch):
            if drain:
                pltpu.make_async_copy(z_hbm, drain_v, sem_s).wait()
            sem = sem_i0 if q % 2 == 0 else sem_i1
            pltpu.make_async_copy(ei_hbm.at[0], idx_v.at[q], sem).wait()
            if prefetch:
                fire_idx((q + 2) % 4, g + 2)
            for b in range(NBC):
                pltpu.async_copy(ones_v, acc_sh.at[idx_v.at[q, b, 1]],
                                 sem_s, add=True)

        fire_idx(0, 0)
        fire_idx(1, 1)
        run_group(0, 0, False, True)
        run_group(1, 1, False, True)

        def body(m, _):
            g = 2 + 4 * m
            run_group(2, g, True, True)
            run_group(3, g + 1, True, True)
            run_group(0, g + 2, True, True)
            run_group(1, g + 3, True, True)
            return 0

        lax.fori_loop(0, (gpt - 5) // 4, body, 0)
        run_group(2, gpt - 3, True, True)
        run_group(3, gpt - 2, True, False)
        run_group(0, gpt - 1, True, False)
        pltpu.make_async_copy(z_hbm, drain_v, sem_s).wait()
        pltpu.make_async_copy(z_hbm, drain_v, sem_s).wait()
        plsc.subcore_barrier()
        for m in range(COPY_PER_TILE):
            sl = pl.ds(s * ROWS_PER_TILE + m * CHUNK, CHUNK)
            pltpu.sync_copy(acc_sh.at[sl], stage_v)
            pltpu.sync_copy(stage_v, out_hbm.at[c, sl])

    return k(ei8, onechunk, z8)


def _layer0_tc(x_ref, wrel_ref, wroot_ref, b_ref, y2_ref, r_ref):
    x0 = x_ref[0]
    x1 = x_ref[1]
    x2 = x_ref[2]
    wr = wrel_ref[...]
    wt = wroot_ref[...]
    dot = functools.partial(jnp.dot, preferred_element_type=jnp.float32)
    y = dot(x0, wr[0:64]) + dot(x1, wr[64:128]) + dot(x2, wr[128:192])
    r = dot(x0, wt[0:64]) + dot(x1, wt[64:128]) + dot(x2, wt[128:192])
    y2_ref[0] = y[:, :32]
    y2_ref[1] = y[:, 32:]
    r_ref[...] = r + b_ref[...]


def _elu_mean(s_ref, cnt_ref, r_ref):
    cnt = cnt_ref[0, :, 0:1] + cnt_ref[1, :, 0:1]
    inv = 1.0 / jnp.maximum(cnt, 1.0)
    ssum = jnp.concatenate([s_ref[0], s_ref[1]], axis=1)
    h = ssum * inv + r_ref[...]
    return jnp.where(h > 0, h, jnp.exp(h) - 1.0)


def _layer1_tc(s_ref, cnt_ref, r0_ref, wrel_ref, wroot_ref, b_ref,
               y2_ref, r_ref):
    h = _elu_mean(s_ref, cnt_ref, r0_ref)
    dot = functools.partial(jnp.dot, preferred_element_type=jnp.float32)
    y = dot(h, wrel_ref[...])
    y2_ref[0] = y[:, :32]
    y2_ref[1] = y[:, 32:]
    r_ref[...] = dot(h, wroot_ref[...]) + b_ref[...]


def _final_tc(s_ref, cnt_ref, r1_ref, wout_ref, bout_ref, out_ref):
    h = _elu_mean(s_ref, cnt_ref, r1_ref)
    out_ref[...] = jnp.dot(h, wout_ref[...],
                           preferred_element_type=jnp.float32) + bout_ref[...]


def _layer0_call(x, wrel, wroot, b):
    return pl.pallas_call(
        _layer0_tc,
        grid=(NP // BLK,),
        in_specs=[
            pl.BlockSpec((3, BLK, 64), lambda i: (0, i, 0)),
            pl.BlockSpec((192, 64), lambda i: (0, 0)),
            pl.BlockSpec((192, 64), lambda i: (0, 0)),
            pl.BlockSpec((1, 64), lambda i: (0, 0)),
        ],
        out_specs=[
            pl.BlockSpec((2, BLK, W), lambda i: (0, i, 0)),
            pl.BlockSpec((BLK, 64), lambda i: (i, 0)),
        ],
        out_shape=[
            jax.ShapeDtypeStruct((2, NP, W), jnp.float32),
            jax.ShapeDtypeStruct((NP, 64), jnp.float32),
        ],
    )(x, wrel, wroot, b)


def _layer1_call(s, cnt, r0, wrel, wroot, b):
    return pl.pallas_call(
        _layer1_tc,
        grid=(NP // BLK,),
        in_specs=[
            pl.BlockSpec((2, BLK, W), lambda i: (0, i, 0)),
            pl.BlockSpec((2, BLK, W), lambda i: (0, i, 0)),
            pl.BlockSpec((BLK, 64), lambda i: (i, 0)),
            pl.BlockSpec((64, 64), lambda i: (0, 0)),
            pl.BlockSpec((64, 64), lambda i: (0, 0)),
            pl.BlockSpec((1, 64), lambda i: (0, 0)),
        ],
        out_specs=[
            pl.BlockSpec((2, BLK, W), lambda i: (0, i, 0)),
            pl.BlockSpec((BLK, 64), lambda i: (i, 0)),
        ],
        out_shape=[
            jax.ShapeDtypeStruct((2, NP, W), jnp.float32),
            jax.ShapeDtypeStruct((NP, 64), jnp.float32),
        ],
    )(s, cnt, r0, wrel, wroot, b)


def _final_call(s, cnt, r1, wout, bout):
    return pl.pallas_call(
        _final_tc,
        grid=(NP // BLK,),
        in_specs=[
            pl.BlockSpec((2, BLK, W), lambda i: (0, i, 0)),
            pl.BlockSpec((2, BLK, W), lambda i: (0, i, 0)),
            pl.BlockSpec((BLK, 64), lambda i: (i, 0)),
            pl.BlockSpec((64, 32), lambda i: (0, 0)),
            pl.BlockSpec((1, 32), lambda i: (0, 0)),
        ],
        out_specs=pl.BlockSpec((BLK, 32), lambda i: (i, 0)),
        out_shape=jax.ShapeDtypeStruct((NP, 32), jnp.float32),
    )(s, cnt, r1, wout, bout)


def kernel(x, edge_index, emb0, emb1, emb2, W_rel0, W_root0, b0,
           W_rel1, W_root1, b1, W_out, b_out):
    n = x.shape[0]
    e = edge_index.shape[1]

    xidx = jnp.pad(x.astype(jnp.int32).T, ((0, 0), (0, NP - n)))
    xidx = xidx.reshape(3, NP // CHUNK, CHUNK)
    xidx = jnp.concatenate(
        [xidx, jnp.broadcast_to(xidx[:, -1:], (3, 32 * XPT - NP // CHUNK,
                                               CHUNK))], axis=1)
    pad_e = EC * CHUNK - e
    fill = jnp.arange(pad_e, dtype=jnp.int32)
    srcp = jnp.concatenate([edge_index[0].astype(jnp.int32), fill % n])
    dstp = jnp.concatenate([edge_index[1].astype(jnp.int32),
                            n + fill % (NP - n)])
    ei = jnp.stack([srcp.reshape(EC, CHUNK), dstp.reshape(EC, CHUNK)],
                   axis=1)
    ei4 = ei.reshape(EC // NB, NB, 2, CHUNK)
    ei8 = ei.reshape(CGROUPS, NBC, 2, CHUNK)
    zrows = jnp.zeros((NB, CHUNK, W), jnp.float32)
    z8 = jnp.zeros((NBC, CHUNK, W), jnp.float32)
    onechunk = jnp.zeros((CHUNK, W), jnp.float32).at[:, 0].set(1.0)

    X = _embed_call(xidx, emb0, emb1, emb2)
    cnt = _cnt_call(ei8, onechunk, z8)

    y20, r0 = _layer0_call(X, W_rel0, W_root0, b0.reshape(1, 64))
    s0 = _agg_call(y20.reshape(2 * NP, W), ei4, zrows)

    y21, r1 = _layer1_call(s0, cnt, r0, W_rel1, W_root1, b1.reshape(1, 64))
    s1 = _agg_call(y21.reshape(2 * NP, W), ei4, zrows)

    logits = _final_call(s1, cnt, r1, W_out, b_out.reshape(1, 32))
    return logits[:n]


# speedup vs baseline: 1.2111x; 1.2096x over previous
import functools

import jax
import jax.numpy as jnp
from jax import lax
from jax.experimental import pallas as pl
from jax.experimental.pallas import tpu as pltpu
from jax.experimental.pallas import tpu_sc as plsc

NP = 51200
CHUNK = 128
W = 32
BLK = 1024
N_TILES = 16
ROWS_PER_TILE = NP // N_TILES
COPY_PER_TILE = ROWS_PER_TILE // CHUNK
ECHUNKS = 6250


XCH = NP // CHUNK
XPT = (XCH + 31) // 32


def _embed_call(xidx, e0, e1, e2):
    mesh = plsc.VectorSubcoreMesh(core_axis_name="c", subcore_axis_name="s")

    @functools.partial(
        pl.kernel, mesh=mesh,
        out_type=jax.ShapeDtypeStruct((3, NP, 64), jnp.float32),
        compiler_params=pltpu.CompilerParams(use_tc_tiling_on_sc=False),
        scratch_types=[
            pltpu.VMEM((XPT, CHUNK), jnp.int32),
            pltpu.VMEM((2, CHUNK, 64), jnp.float32),
            pltpu.VMEM_SHARED((3, 1024, 64), jnp.float32),
            pltpu.SemaphoreType.DMA,
            pltpu.SemaphoreType.DMA,
        ],
    )
    def k(xidx_hbm, e0_hbm, e1_hbm, e2_hbm, out_hbm, idx_v, rows_v,
          tab_sh, sem_g, sem_w):
        c = lax.axis_index("c")
        s = lax.axis_index("s")
        wid = s * 2 + c
        base = wid * XPT
        tabs = (e0_hbm, e1_hbm, e2_hbm)
        tsl = pl.ds(s * 64, 64)
        for f in range(3):
            pltpu.sync_copy(tabs[f].at[tsl], rows_v.at[0, pl.ds(0, 64)])
            pltpu.sync_copy(rows_v.at[0, pl.ds(0, 64)], tab_sh.at[f, tsl])
        plsc.subcore_barrier()
        zdesc = lambda p: pltpu.make_async_copy(
            out_hbm.at[0, pl.ds(0, CHUNK)], rows_v.at[p], sem_w)

        for f in range(3):
            pltpu.sync_copy(xidx_hbm.at[f, pl.ds(base, XPT)], idx_v)

            def gather(p, j, f=f):
                pltpu.async_copy(tab_sh.at[f].at[idx_v.at[j]],
                                 rows_v.at[p], sem_g)
                pltpu.make_async_copy(out_hbm.at[0, pl.ds(0, CHUNK)],
                                      rows_v.at[p], sem_g).wait()

            def writeout(p, j, f=f):
                ch = jnp.minimum(base + j, XCH - 1)
                pltpu.async_copy(rows_v.at[p],
                                 out_hbm.at[f, pl.ds(ch * CHUNK, CHUNK)],
                                 sem_w)

            for j in range(XPT):
                p = j % 2
                if j >= 2:
                    zdesc(p).wait()
                gather(p, j)
                writeout(p, j)
            zdesc((XPT - 2) % 2).wait()
            zdesc((XPT - 1) % 2).wait()

    return k(xidx, e0, e1, e2)


NB = 2
EC = 6400
EGROUPS = EC // NB
G_PER_TILE = EGROUPS // N_TILES


def _agg_call(y2, ei4, zrows):
    mesh = plsc.VectorSubcoreMesh(core_axis_name="c", subcore_axis_name="s")

    @functools.partial(
        pl.kernel, mesh=mesh,
        out_type=jax.ShapeDtypeStruct((2, NP, W), jnp.float32),
        compiler_params=pltpu.CompilerParams(use_tc_tiling_on_sc=False),
        scratch_types=[
            pltpu.VMEM((4, NB, 2, CHUNK), jnp.int32),
            pltpu.VMEM((4, NB, CHUNK), jnp.int32),
            pltpu.VMEM((2, NB, CHUNK, W), jnp.float32),
            pltpu.VMEM((CHUNK, W), jnp.float32),
            pltpu.VMEM_SHARED((NP, W), jnp.float32),
            pltpu.SemaphoreType.DMA,
            pltpu.SemaphoreType.DMA,
            pltpu.SemaphoreType.DMA,
            pltpu.SemaphoreType.DMA,
        ],
    )
    def k(y2_hbm, ei_hbm, z_hbm, out_hbm,
          idx_v, soff_v, rows_v, stage_v, acc_sh, sem_i0, sem_i1,
          sem_g, sem_s):
        c = lax.axis_index("c")
        s = lax.axis_index("s")
        coff = c * NP
        pltpu.sync_copy(z_hbm.at[0], stage_v)
        for m in range(COPY_PER_TILE):
            pltpu.sync_copy(stage_v,
                            acc_sh.at[pl.ds(s * ROWS_PER_TILE + m * CHUNK,
                                            CHUNK)])
        plsc.subcore_barrier()
        gbase = s * G_PER_TILE

        def fire_idx(q, g):
            sem = sem_i0 if q % 2 == 0 else sem_i1
            pltpu.async_copy(ei_hbm.at[gbase + g], idx_v.at[q], sem)

        def run_group(p, q, g, drain, prefetch):
            if drain:
                pltpu.make_async_copy(z_hbm, rows_v.at[p], sem_s).wait()
            sem = sem_i0 if q % 2 == 0 else sem_i1
            pltpu.make_async_copy(ei_hbm.at[0], idx_v.at[q], sem).wait()
            if prefetch:
                fire_idx((q + 2) % 4, g + 2)
            for b in range(NB):
                for i in range(CHUNK // 16):
                    sl = pl.ds(i * 16, 16)
                    soff_v[q, b, sl] = idx_v[q, b, 0, sl] + coff
            for b in range(NB):
                pltpu.async_copy(y2_hbm.at[soff_v.at[q, b]],
                                 rows_v.at[p, b], sem_g)
            pltpu.make_async_copy(z_hbm, rows_v.at[p], sem_g).wait()
            for b in range(NB):
                pltpu.async_copy(rows_v.at[p, b],
                                 acc_sh.at[idx_v.at[q, b, 1]], sem_s,
                                 add=True)

        fire_idx(0, 0)
        fire_idx(1, 1)
        run_group(0, 0, 0, False, True)
        run_group(1, 1, 1, False, True)

        def body(m, _):
            g = 2 + 4 * m
            run_group(0, 2, g, True, True)
            run_group(1, 3, g + 1, True, True)
            run_group(0, 0, g + 2, True, True)
            run_group(1, 1, g + 3, True, True)
            return 0

        lax.fori_loop(0, (G_PER_TILE - 4) // 4, body, 0)
        run_group(0, 2, G_PER_TILE - 2, True, False)
        run_group(1, 3, G_PER_TILE - 1, True, False)
        pltpu.make_async_copy(z_hbm, rows_v.at[0], sem_s).wait()
        pltpu.make_async_copy(z_hbm, rows_v.at[1], sem_s).wait()
        plsc.subcore_barrier()
        for m in range(COPY_PER_TILE):
            sl = pl.ds(s * ROWS_PER_TILE + m * CHUNK, CHUNK)
            pltpu.sync_copy(acc_sh.at[sl], stage_v)
            pltpu.sync_copy(stage_v, out_hbm.at[c, sl])

    return k(y2, ei4, zrows)


NBC = 8
CGROUPS = EC // NBC
CG_PER_TILE = CGROUPS // 2 // N_TILES


def _cnt_call(ei8, onechunk, z8):
    mesh = plsc.VectorSubcoreMesh(core_axis_name="c", subcore_axis_name="s")
    gpt = CG_PER_TILE

    @functools.partial(
        pl.kernel, mesh=mesh,
        out_type=jax.ShapeDtypeStruct((2, NP, W), jnp.float32),
        compiler_params=pltpu.CompilerParams(use_tc_tiling_on_sc=False),
        scratch_types=[
            pltpu.VMEM((4, NBC, 2, CHUNK), jnp.int32),
            pltpu.VMEM((CHUNK, W), jnp.float32),
            pltpu.VMEM((CHUNK, W), jnp.float32),
            pltpu.VMEM((NBC, CHUNK, W), jnp.float32),
            pltpu.VMEM_SHARED((NP, W), jnp.float32),
            pltpu.SemaphoreType.DMA,
            pltpu.SemaphoreType.DMA,
            pltpu.SemaphoreType.DMA,
        ],
    )
    def k(ei_hbm, one_hbm, z_hbm, out_hbm,
          idx_v, ones_v, stage_v, drain_v, acc_sh, sem_i0, sem_i1, sem_s):
        c = lax.axis_index("c")
        s = lax.axis_index("s")
        pltpu.sync_copy(z_hbm.at[0], stage_v)
        for m in range(COPY_PER_TILE):
            pltpu.sync_copy(stage_v,
                            acc_sh.at[pl.ds(s * ROWS_PER_TILE + m * CHUNK,
                                            CHUNK)])
        pltpu.sync_copy(one_hbm, ones_v)
        plsc.subcore_barrier()
        gbase = (c * N_TILES + s) * gpt

        def fire_idx(q, g):
            sem = sem_i0 if q % 2 == 0 else sem_i1
            pltpu.async_copy(ei_hbm.at[gbase + g], idx_v.at[q], sem)

        def run_group(q, g, drain, prefetch):
            if drain:
                pltpu.make_async_copy(z_hbm, drain_v, sem_s).wait()
            sem = sem_i0 if q % 2 == 0 else sem_i1
            pltpu.make_async_copy(ei_hbm.at[0], idx_v.at[q], sem).wait()
            if prefetch:
                fire_idx((q + 2) % 4, g + 2)
            for b in range(NBC):
                pltpu.async_copy(ones_v, acc_sh.at[idx_v.at[q, b, 1]],
                                 sem_s, add=True)

        fire_idx(0, 0)
        fire_idx(1, 1)
        run_group(0, 0, False, True)
        run_group(1, 1, False, True)

        def body(m, _):
            g = 2 + 4 * m
            run_group(2, g, True, True)
            run_group(3, g + 1, True, True)
            run_group(0, g + 2, True, True)
            run_group(1, g + 3, True, True)
            return 0

        lax.fori_loop(0, (gpt - 5) // 4, body, 0)
        run_group(2, gpt - 3, True, True)
        run_group(3, gpt - 2, True, False)
        run_group(0, gpt - 1, True, False)
        pltpu.make_async_copy(z_hbm, drain_v, sem_s).wait()
        pltpu.make_async_copy(z_hbm, drain_v, sem_s).wait()
        plsc.subcore_barrier()
        for m in range(COPY_PER_TILE):
            sl = pl.ds(s * ROWS_PER_TILE + m * CHUNK, CHUNK)
            pltpu.sync_copy(acc_sh.at[sl], stage_v)
            pltpu.sync_copy(stage_v, out_hbm.at[c, sl])

    return k(ei8, onechunk, z8)


def _layer0_tc(x_ref, wrel_ref, wroot_ref, b_ref, y2_ref, r_ref):
    x0 = x_ref[0]
    x1 = x_ref[1]
    x2 = x_ref[2]
    wr = wrel_ref[...]
    wt = wroot_ref[...]
    dot = functools.partial(jnp.dot, preferred_element_type=jnp.float32)
    y = dot(x0, wr[0:64]) + dot(x1, wr[64:128]) + dot(x2, wr[128:192])
    r = dot(x0, wt[0:64]) + dot(x1, wt[64:128]) + dot(x2, wt[128:192])
    y2_ref[0] = y[:, :32]
    y2_ref[1] = y[:, 32:]
    r_ref[...] = r + b_ref[...]


def _elu_mean(s_ref, cnt_ref, r_ref):
    cnt = cnt_ref[0, :, 0:1] + cnt_ref[1, :, 0:1]
    inv = 1.0 / jnp.maximum(cnt, 1.0)
    ssum = jnp.concatenate([s_ref[0], s_ref[1]], axis=1)
    h = ssum * inv + r_ref[...]
    return jnp.where(h > 0, h, jnp.exp(h) - 1.0)


def _layer1_tc(s_ref, cnt_ref, r0_ref, wrel_ref, wroot_ref, b_ref,
               y2_ref, r_ref):
    h = _elu_mean(s_ref, cnt_ref, r0_ref)
    dot = functools.partial(jnp.dot, preferred_element_type=jnp.float32)
    y = dot(h, wrel_ref[...])
    y2_ref[0] = y[:, :32]
    y2_ref[1] = y[:, 32:]
    r_ref[...] = dot(h, wroot_ref[...]) + b_ref[...]


def _final_tc(s_ref, cnt_ref, r1_ref, wout_ref, bout_ref, out_ref):
    h = _elu_mean(s_ref, cnt_ref, r1_ref)
    out_ref[...] = jnp.dot(h, wout_ref[...],
                           preferred_element_type=jnp.float32) + bout_ref[...]


def _layer0_call(x, wrel, wroot, b):
    return pl.pallas_call(
        _layer0_tc,
        grid=(NP // BLK,),
        in_specs=[
            pl.BlockSpec((3, BLK, 64), lambda i: (0, i, 0)),
            pl.BlockSpec((192, 64), lambda i: (0, 0)),
            pl.BlockSpec((192, 64), lambda i: (0, 0)),
            pl.BlockSpec((1, 64), lambda i: (0, 0)),
        ],
        out_specs=[
            pl.BlockSpec((2, BLK, W), lambda i: (0, i, 0)),
            pl.BlockSpec((BLK, 64), lambda i: (i, 0)),
        ],
        out_shape=[
            jax.ShapeDtypeStruct((2, NP, W), jnp.float32),
            jax.ShapeDtypeStruct((NP, 64), jnp.float32),
        ],
    )(x, wrel, wroot, b)


def _layer1_call(s, cnt, r0, wrel, wroot, b):
    return pl.pallas_call(
        _layer1_tc,
        grid=(NP // BLK,),
        in_specs=[
            pl.BlockSpec((2, BLK, W), lambda i: (0, i, 0)),
            pl.BlockSpec((2, BLK, W), lambda i: (0, i, 0)),
            pl.BlockSpec((BLK, 64), lambda i: (i, 0)),
            pl.BlockSpec((64, 64), lambda i: (0, 0)),
            pl.BlockSpec((64, 64), lambda i: (0, 0)),
            pl.BlockSpec((1, 64), lambda i: (0, 0)),
        ],
        out_specs=[
            pl.BlockSpec((2, BLK, W), lambda i: (0, i, 0)),
            pl.BlockSpec((BLK, 64), lambda i: (i, 0)),
        ],
        out_shape=[
            jax.ShapeDtypeStruct((2, NP, W), jnp.float32),
            jax.ShapeDtypeStruct((NP, 64), jnp.float32),
        ],
    )(s, cnt, r0, wrel, wroot, b)


def _final_call(s, cnt, r1, wout, bout):
    return pl.pallas_call(
        _final_tc,
        grid=(NP // BLK,),
        in_specs=[
            pl.BlockSpec((2, BLK, W), lambda i: (0, i, 0)),
            pl.BlockSpec((2, BLK, W), lambda i: (0, i, 0)),
            pl.BlockSpec((BLK, 64), lambda i: (i, 0)),
            pl.BlockSpec((64, 32), lambda i: (0, 0)),
            pl.BlockSpec((1, 32), lambda i: (0, 0)),
        ],
        out_specs=pl.BlockSpec((BLK, 32), lambda i: (i, 0)),
        out_shape=jax.ShapeDtypeStruct((NP, 32), jnp.float32),
    )(s, cnt, r1, wout, bout)


def kernel(x, edge_index, emb0, emb1, emb2, W_rel0, W_root0, b0,
           W_rel1, W_root1, b1, W_out, b_out):
    n = x.shape[0]
    e = edge_index.shape[1]

    xidx = jnp.pad(x.astype(jnp.int32).T, ((0, 0), (0, NP - n)))
    xidx = xidx.reshape(3, NP // CHUNK, CHUNK)
    xidx = jnp.concatenate(
        [xidx, jnp.broadcast_to(xidx[:, -1:], (3, 32 * XPT - NP // CHUNK,
                                               CHUNK))], axis=1)
    pad_e = EC * CHUNK - e
    fill = jnp.arange(pad_e, dtype=jnp.int32)
    srcp = jnp.concatenate([edge_index[0].astype(jnp.int32), fill % n])
    dstp = jnp.concatenate([edge_index[1].astype(jnp.int32),
                            n + fill % (NP - n)])
    ei = jnp.stack([srcp.reshape(EC, CHUNK), dstp.reshape(EC, CHUNK)],
                   axis=1)
    ei4 = ei.reshape(EC // NB, NB, 2, CHUNK)
    ei8 = ei.reshape(CGROUPS, NBC, 2, CHUNK)
    zrows = jnp.zeros((NB, CHUNK, W), jnp.float32)
    z8 = jnp.zeros((NBC, CHUNK, W), jnp.float32)
    onechunk = jnp.zeros((CHUNK, W), jnp.float32).at[:, 0].set(1.0)

    tpad = lambda t: jnp.pad(t, ((0, 1024 - t.shape[0]), (0, 0)))
    X = _embed_call(xidx, tpad(emb0), tpad(emb1), tpad(emb2))
    cnt = _cnt_call(ei8, onechunk, z8)

    y20, r0 = _layer0_call(X, W_rel0, W_root0, b0.reshape(1, 64))
    s0 = _agg_call(y20.reshape(2 * NP, W), ei4, zrows)

    y21, r1 = _layer1_call(s0, cnt, r0, W_rel1, W_root1, b1.reshape(1, 64))
    s1 = _agg_call(y21.reshape(2 * NP, W), ei4, zrows)

    logits = _final_call(s1, cnt, r1, W_out, b_out.reshape(1, 32))
    return logits[:n]


# speedup vs baseline: 1.2496x; 1.0318x over previous
import functools

import jax
import jax.numpy as jnp
from jax import lax
from jax.experimental import pallas as pl
from jax.experimental.pallas import tpu as pltpu
from jax.experimental.pallas import tpu_sc as plsc

NP = 51200
CHUNK = 128
W = 32
BLK = 1024
N_TILES = 16
ROWS_PER_TILE = NP // N_TILES
COPY_PER_TILE = ROWS_PER_TILE // CHUNK
ECHUNKS = 6250


XCH = NP // CHUNK
XPT = (XCH + 31) // 32


def _embed_call(xidx, e0, e1, e2):
    mesh = plsc.VectorSubcoreMesh(core_axis_name="c", subcore_axis_name="s")

    @functools.partial(
        pl.kernel, mesh=mesh,
        out_type=jax.ShapeDtypeStruct((3, NP, 64), jnp.float32),
        compiler_params=pltpu.CompilerParams(use_tc_tiling_on_sc=False),
        scratch_types=[
            pltpu.VMEM((XPT, CHUNK), jnp.int32),
            pltpu.VMEM((2, CHUNK, 64), jnp.float32),
            pltpu.VMEM_SHARED((3, 1024, 64), jnp.float32),
            pltpu.SemaphoreType.DMA,
            pltpu.SemaphoreType.DMA,
        ],
    )
    def k(xidx_hbm, e0_hbm, e1_hbm, e2_hbm, out_hbm, idx_v, rows_v,
          tab_sh, sem_g, sem_w):
        c = lax.axis_index("c")
        s = lax.axis_index("s")
        wid = s * 2 + c
        base = wid * XPT
        tabs = (e0_hbm, e1_hbm, e2_hbm)
        tsl = pl.ds(s * 64, 64)
        for f in range(3):
            pltpu.sync_copy(tabs[f].at[tsl], rows_v.at[0, pl.ds(0, 64)])
            pltpu.sync_copy(rows_v.at[0, pl.ds(0, 64)], tab_sh.at[f, tsl])
        plsc.subcore_barrier()
        zdesc = lambda p: pltpu.make_async_copy(
            out_hbm.at[0, pl.ds(0, CHUNK)], rows_v.at[p], sem_w)

        for f in range(3):
            pltpu.sync_copy(xidx_hbm.at[f, pl.ds(base, XPT)], idx_v)

            def gather(p, j, f=f):
                pltpu.async_copy(tab_sh.at[f].at[idx_v.at[j]],
                                 rows_v.at[p], sem_g)
                pltpu.make_async_copy(out_hbm.at[0, pl.ds(0, CHUNK)],
                                      rows_v.at[p], sem_g).wait()

            def writeout(p, j, f=f):
                ch = jnp.minimum(base + j, XCH - 1)
                pltpu.async_copy(rows_v.at[p],
                                 out_hbm.at[f, pl.ds(ch * CHUNK, CHUNK)],
                                 sem_w)

            for j in range(XPT):
                p = j % 2
                if j >= 2:
                    zdesc(p).wait()
                gather(p, j)
                writeout(p, j)
            zdesc((XPT - 2) % 2).wait()
            zdesc((XPT - 1) % 2).wait()

    return k(xidx, e0, e1, e2)


NB = 1
EC = 6400
EGROUPS = EC // NB
G_PER_TILE = EGROUPS // N_TILES


def _agg_call(y2, ei4, zrows):
    mesh = plsc.VectorSubcoreMesh(core_axis_name="c", subcore_axis_name="s")

    @functools.partial(
        pl.kernel, mesh=mesh,
        out_type=jax.ShapeDtypeStruct((2, NP, W), jnp.float32),
        compiler_params=pltpu.CompilerParams(use_tc_tiling_on_sc=False),
        scratch_types=[
            pltpu.VMEM((4, NB, 2, CHUNK), jnp.int32),
            pltpu.VMEM((4, NB, CHUNK), jnp.int32),
            pltpu.VMEM((4, NB, CHUNK, W), jnp.float32),
            pltpu.VMEM((CHUNK, W), jnp.float32),
            pltpu.VMEM_SHARED((NP, W), jnp.float32),
            pltpu.SemaphoreType.DMA,
            pltpu.SemaphoreType.DMA,
            pltpu.SemaphoreType.DMA,
            pltpu.SemaphoreType.DMA,
        ],
    )
    def k(y2_hbm, ei_hbm, z_hbm, out_hbm,
          idx_v, soff_v, rows_v, stage_v, acc_sh, sem_i0, sem_i1,
          sem_g, sem_s):
        c = lax.axis_index("c")
        s = lax.axis_index("s")
        coff = c * NP
        pltpu.sync_copy(z_hbm.at[0], stage_v)
        for m in range(COPY_PER_TILE):
            pltpu.sync_copy(stage_v,
                            acc_sh.at[pl.ds(s * ROWS_PER_TILE + m * CHUNK,
                                            CHUNK)])
        plsc.subcore_barrier()
        gbase = s * G_PER_TILE

        def fire_idx(q, g):
            sem = sem_i0 if q % 2 == 0 else sem_i1
            pltpu.async_copy(ei_hbm.at[g], idx_v.at[q], sem)

        def wait_idx(q):
            sem = sem_i0 if q % 2 == 0 else sem_i1
            pltpu.make_async_copy(ei_hbm.at[0], idx_v.at[q], sem).wait()

        def soff(q):
            for b in range(NB):
                for i in range(CHUNK // 16):
                    sl = pl.ds(i * 16, 16)
                    soff_v[q, b, sl] = idx_v[q, b, 0, sl] + coff

        def fire_gathers(q):
            for b in range(NB):
                pltpu.async_copy(y2_hbm.at[soff_v.at[q, b]],
                                 rows_v.at[q, b], sem_g)

        def step(q, g, drain, fire2, nxt):
            if drain:
                pltpu.make_async_copy(z_hbm, rows_v.at[q], sem_s).wait()
            if fire2:
                fire_idx((q + 2) % 4, g + 2)
            if nxt:
                wait_idx((q + 1) % 4)
                soff((q + 1) % 4)
                fire_gathers((q + 1) % 4)
            pltpu.make_async_copy(z_hbm, rows_v.at[q], sem_g).wait()
            for b in range(NB):
                pltpu.async_copy(rows_v.at[q, b],
                                 acc_sh.at[idx_v.at[q, b, 1]], sem_s,
                                 add=True)

        fire_idx(0, gbase)
        fire_idx(1, gbase + 1)
        wait_idx(0)
        soff(0)
        fire_gathers(0)
        step(0, gbase, False, True, True)
        step(1, gbase + 1, False, True, True)

        def body(m, _):
            g = gbase + 2 + 4 * m
            step(2, g, True, True, True)
            step(3, g + 1, True, True, True)
            step(0, g + 2, True, True, True)
            step(1, g + 3, True, True, True)
            return 0

        lax.fori_loop(0, (G_PER_TILE - 4) // 4, body, 0)
        step(2, gbase + G_PER_TILE - 2, True, False, True)
        step(3, gbase + G_PER_TILE - 1, True, False, False)
        pltpu.make_async_copy(z_hbm, rows_v.at[0], sem_s).wait()
        pltpu.make_async_copy(z_hbm, rows_v.at[1], sem_s).wait()
        plsc.subcore_barrier()
        for m in range(COPY_PER_TILE):
            sl = pl.ds(s * ROWS_PER_TILE + m * CHUNK, CHUNK)
            pltpu.sync_copy(acc_sh.at[sl], stage_v)
            pltpu.sync_copy(stage_v, out_hbm.at[c, sl])

    return k(y2, ei4, zrows)


NBC = 8
CGROUPS = EC // NBC
CG_PER_TILE = CGROUPS // 2 // N_TILES


def _cnt_call(ei8, onechunk, z8):
    mesh = plsc.VectorSubcoreMesh(core_axis_name="c", subcore_axis_name="s")
    gpt = CG_PER_TILE

    @functools.partial(
        pl.kernel, mesh=mesh,
        out_type=jax.ShapeDtypeStruct((2, NP, W), jnp.float32),
        compiler_params=pltpu.CompilerParams(use_tc_tiling_on_sc=False),
        scratch_types=[
            pltpu.VMEM((4, NBC, 2, CHUNK), jnp.int32),
            pltpu.VMEM((CHUNK, W), jnp.float32),
            pltpu.VMEM((CHUNK, W), jnp.float32),
            pltpu.VMEM((NBC, CHUNK, W), jnp.float32),
            pltpu.VMEM_SHARED((NP, W), jnp.float32),
            pltpu.SemaphoreType.DMA,
            pltpu.SemaphoreType.DMA,
            pltpu.SemaphoreType.DMA,
        ],
    )
    def k(ei_hbm, one_hbm, z_hbm, out_hbm,
          idx_v, ones_v, stage_v, drain_v, acc_sh, sem_i0, sem_i1, sem_s):
        c = lax.axis_index("c")
        s = lax.axis_index("s")
        pltpu.sync_copy(z_hbm.at[0], stage_v)
        for m in range(COPY_PER_TILE):
            pltpu.sync_copy(stage_v,
                            acc_sh.at[pl.ds(s * ROWS_PER_TILE + m * CHUNK,
                                            CHUNK)])
        pltpu.sync_copy(one_hbm, ones_v)
        plsc.subcore_barrier()
        gbase = (c * N_TILES + s) * gpt

        def fire_idx(q, g):
            sem = sem_i0 if q % 2 == 0 else sem_i1
            pltpu.async_copy(ei_hbm.at[gbase + g], idx_v.at[q], sem)

        def run_group(q, g, drain, prefetch):
            if drain:
                pltpu.make_async_copy(z_hbm, drain_v, sem_s).wait()
            sem = sem_i0 if q % 2 == 0 else sem_i1
            pltpu.make_async_copy(ei_hbm.at[0], idx_v.at[q], sem).wait()
            if prefetch:
                fire_idx((q + 2) % 4, g + 2)
            for b in range(NBC):
                pltpu.async_copy(ones_v, acc_sh.at[idx_v.at[q, b, 1]],
                                 sem_s, add=True)

        fire_idx(0, 0)
        fire_idx(1, 1)
        run_group(0, 0, False, True)
        run_group(1, 1, False, True)

        def body(m, _):
            g = 2 + 4 * m
            run_group(2, g, True, True)
            run_group(3, g + 1, True, True)
            run_group(0, g + 2, True, True)
            run_group(1, g + 3, True, True)
            return 0

        lax.fori_loop(0, (gpt - 5) // 4, body, 0)
        run_group(2, gpt - 3, True, True)
        run_group(3, gpt - 2, True, False)
        run_group(0, gpt - 1, True, False)
        pltpu.make_async_copy(z_hbm, drain_v, sem_s).wait()
        pltpu.make_async_copy(z_hbm, drain_v, sem_s).wait()
        plsc.subcore_barrier()
        for m in range(COPY_PER_TILE):
            sl = pl.ds(s * ROWS_PER_TILE + m * CHUNK, CHUNK)
            pltpu.sync_copy(acc_sh.at[sl], stage_v)
            pltpu.sync_copy(stage_v, out_hbm.at[c, sl])

    return k(ei8, onechunk, z8)


def _layer0_tc(x_ref, wrel_ref, wroot_ref, b_ref, y2_ref, r_ref):
    x0 = x_ref[0]
    x1 = x_ref[1]
    x2 = x_ref[2]
    wr = wrel_ref[...]
    wt = wroot_ref[...]
    dot = functools.partial(jnp.dot, preferred_element_type=jnp.float32)
    y = dot(x0, wr[0:64]) + dot(x1, wr[64:128]) + dot(x2, wr[128:192])
    r = dot(x0, wt[0:64]) + dot(x1, wt[64:128]) + dot(x2, wt[128:192])
    y2_ref[0] = y[:, :32]
    y2_ref[1] = y[:, 32:]
    r_ref[...] = r + b_ref[...]


def _elu_mean(s_ref, cnt_ref, r_ref):
    cnt = cnt_ref[0, :, 0:1] + cnt_ref[1, :, 0:1]
    inv = 1.0 / jnp.maximum(cnt, 1.0)
    ssum = jnp.concatenate([s_ref[0], s_ref[1]], axis=1)
    h = ssum * inv + r_ref[...]
    return jnp.where(h > 0, h, jnp.exp(h) - 1.0)


def _layer1_tc(s_ref, cnt_ref, r0_ref, wrel_ref, wroot_ref, b_ref,
               y2_ref, r_ref):
    h = _elu_mean(s_ref, cnt_ref, r0_ref)
    dot = functools.partial(jnp.dot, preferred_element_type=jnp.float32)
    y = dot(h, wrel_ref[...])
    y2_ref[0] = y[:, :32]
    y2_ref[1] = y[:, 32:]
    r_ref[...] = dot(h, wroot_ref[...]) + b_ref[...]


def _final_tc(s_ref, cnt_ref, r1_ref, wout_ref, bout_ref, out_ref):
    h = _elu_mean(s_ref, cnt_ref, r1_ref)
    out_ref[...] = jnp.dot(h, wout_ref[...],
                           preferred_element_type=jnp.float32) + bout_ref[...]


def _layer0_call(x, wrel, wroot, b):
    return pl.pallas_call(
        _layer0_tc,
        grid=(NP // BLK,),
        in_specs=[
            pl.BlockSpec((3, BLK, 64), lambda i: (0, i, 0)),
            pl.BlockSpec((192, 64), lambda i: (0, 0)),
            pl.BlockSpec((192, 64), lambda i: (0, 0)),
            pl.BlockSpec((1, 64), lambda i: (0, 0)),
        ],
        out_specs=[
            pl.BlockSpec((2, BLK, W), lambda i: (0, i, 0)),
            pl.BlockSpec((BLK, 64), lambda i: (i, 0)),
        ],
        out_shape=[
            jax.ShapeDtypeStruct((2, NP, W), jnp.float32),
            jax.ShapeDtypeStruct((NP, 64), jnp.float32),
        ],
    )(x, wrel, wroot, b)


def _layer1_call(s, cnt, r0, wrel, wroot, b):
    return pl.pallas_call(
        _layer1_tc,
        grid=(NP // BLK,),
        in_specs=[
            pl.BlockSpec((2, BLK, W), lambda i: (0, i, 0)),
            pl.BlockSpec((2, BLK, W), lambda i: (0, i, 0)),
            pl.BlockSpec((BLK, 64), lambda i: (i, 0)),
            pl.BlockSpec((64, 64), lambda i: (0, 0)),
            pl.BlockSpec((64, 64), lambda i: (0, 0)),
            pl.BlockSpec((1, 64), lambda i: (0, 0)),
        ],
        out_specs=[
            pl.BlockSpec((2, BLK, W), lambda i: (0, i, 0)),
            pl.BlockSpec((BLK, 64), lambda i: (i, 0)),
        ],
        out_shape=[
            jax.ShapeDtypeStruct((2, NP, W), jnp.float32),
            jax.ShapeDtypeStruct((NP, 64), jnp.float32),
        ],
    )(s, cnt, r0, wrel, wroot, b)


def _final_call(s, cnt, r1, wout, bout):
    return pl.pallas_call(
        _final_tc,
        grid=(NP // BLK,),
        in_specs=[
            pl.BlockSpec((2, BLK, W), lambda i: (0, i, 0)),
            pl.BlockSpec((2, BLK, W), lambda i: (0, i, 0)),
            pl.BlockSpec((BLK, 64), lambda i: (i, 0)),
            pl.BlockSpec((64, 32), lambda i: (0, 0)),
            pl.BlockSpec((1, 32), lambda i: (0, 0)),
        ],
        out_specs=pl.BlockSpec((BLK, 32), lambda i: (i, 0)),
        out_shape=jax.ShapeDtypeStruct((NP, 32), jnp.float32),
    )(s, cnt, r1, wout, bout)


def kernel(x, edge_index, emb0, emb1, emb2, W_rel0, W_root0, b0,
           W_rel1, W_root1, b1, W_out, b_out):
    n = x.shape[0]
    e = edge_index.shape[1]

    xidx = jnp.pad(x.astype(jnp.int32).T, ((0, 0), (0, NP - n)))
    xidx = xidx.reshape(3, NP // CHUNK, CHUNK)
    xidx = jnp.concatenate(
        [xidx, jnp.broadcast_to(xidx[:, -1:], (3, 32 * XPT - NP // CHUNK,
                                               CHUNK))], axis=1)
    pad_e = EC * CHUNK - e
    fill = jnp.arange(pad_e, dtype=jnp.int32)
    srcp = jnp.concatenate([edge_index[0].astype(jnp.int32), fill % n])
    dstp = jnp.concatenate([edge_index[1].astype(jnp.int32),
                            n + fill % (NP - n)])
    ei = jnp.stack([srcp.reshape(EC, CHUNK), dstp.reshape(EC, CHUNK)],
                   axis=1)
    ei4 = ei.reshape(EC // NB, NB, 2, CHUNK)
    ei8 = ei.reshape(CGROUPS, NBC, 2, CHUNK)
    zrows = jnp.zeros((NB, CHUNK, W), jnp.float32)
    z8 = jnp.zeros((NBC, CHUNK, W), jnp.float32)
    onechunk = jnp.zeros((CHUNK, W), jnp.float32).at[:, 0].set(1.0)

    tpad = lambda t: jnp.pad(t, ((0, 1024 - t.shape[0]), (0, 0)))
    X = _embed_call(xidx, tpad(emb0), tpad(emb1), tpad(emb2))
    cnt = _cnt_call(ei8, onechunk, z8)

    y20, r0 = _layer0_call(X, W_rel0, W_root0, b0.reshape(1, 64))
    s0 = _agg_call(y20.reshape(2 * NP, W), ei4, zrows)

    y21, r1 = _layer1_call(s0, cnt, r0, W_rel1, W_root1, b1.reshape(1, 64))
    s1 = _agg_call(y21.reshape(2 * NP, W), ei4, zrows)

    logits = _final_call(s1, cnt, r1, W_out, b_out.reshape(1, 32))
    return logits[:n]
